# Initial kernel scaffold; baseline (speedup 1.0000x reference)
import jax
import jax.numpy as jnp
from jax.experimental import pallas as pl


def kernel(inputs, embedding_weight):
    raise NotImplementedError("write your pallas kernel here")


# speedup vs baseline: 1.1130x; 1.1130x over previous
import jax
import jax.numpy as jnp
from jax.experimental import pallas as pl
from jax.experimental.pallas import tpu as pltpu

NUM_K = 8192
DIM = 32
N_TOK = 16 * 1024
BLK = 256
N_BLKS = N_TOK // BLK
SUB = 1024 // BLK
COMMIT = 0.25


def _vq_body(idx_ref, x_ref, w_ref, enc_ref, q_ref, cnt_ref, loss_ref,
             perp_ref):
    i = pl.program_id(0)
    x = x_ref[...].reshape(BLK, DIM)
    w = w_ref[...]
    idx = idx_ref[...]

    iota = jax.lax.broadcasted_iota(jnp.int32, (BLK, NUM_K), 1)
    enc = (iota == idx).astype(jnp.float32)
    enc_ref[...] = enc

    q = jax.lax.dot_general(
        enc, w, dimension_numbers=(((1,), (0,)), ((), ())),
        preferred_element_type=jnp.float32)
    q_ref[...] = (x + (q - x)).reshape(1, BLK, DIM)

    @pl.when(i == 0)
    def _init():
        cnt_ref[...] = jnp.zeros_like(cnt_ref)
        loss_ref[...] = jnp.zeros_like(loss_ref)
        perp_ref[...] = jnp.zeros_like(perp_ref)

    cnt_ref[...] += jnp.sum(enc, axis=0, keepdims=True)
    loss_ref[...] += jnp.sum((q - x) ** 2)[None, None]

    @pl.when(i == N_BLKS - 1)
    def _fin():
        loss_ref[...] = (COMMIT / (N_TOK * DIM)) * loss_ref[...]
        avg = cnt_ref[...] * (1.0 / N_TOK)
        ent = jnp.sum(avg * jnp.log(avg + 1e-10))
        perp_ref[...] = jnp.exp(-ent)[None, None]


def kernel(inputs, embedding_weight):
    flat = inputs.reshape(-1, DIM)
    distances = (jnp.sum(flat ** 2, axis=1, keepdims=True)
                 + jnp.sum(embedding_weight ** 2, axis=1)
                 - 2.0 * jnp.matmul(flat, embedding_weight.T))
    encoding_indices = jnp.argmin(distances, axis=1)

    idx2 = encoding_indices.astype(jnp.int32).reshape(N_TOK, 1)
    enc, q, cnt, loss, perp = pl.pallas_call(
        _vq_body,
        grid=(N_BLKS,),
        in_specs=[
            pl.BlockSpec((BLK, 1), lambda i: (i, 0)),
            pl.BlockSpec((1, BLK, DIM), lambda i: (i // SUB, i % SUB, 0)),
            pl.BlockSpec((NUM_K, DIM), lambda i: (0, 0)),
        ],
        out_specs=[
            pl.BlockSpec((BLK, NUM_K), lambda i: (i, 0)),
            pl.BlockSpec((1, BLK, DIM), lambda i: (i // SUB, i % SUB, 0)),
            pl.BlockSpec((1, NUM_K), lambda i: (0, 0)),
            pl.BlockSpec((1, 1), lambda i: (0, 0)),
            pl.BlockSpec((1, 1), lambda i: (0, 0)),
        ],
        out_shape=[
            jax.ShapeDtypeStruct((N_TOK, NUM_K), jnp.float32),
            jax.ShapeDtypeStruct((16, 1024, DIM), jnp.float32),
            jax.ShapeDtypeStruct((1, NUM_K), jnp.float32),
            jax.ShapeDtypeStruct((1, 1), jnp.float32),
            jax.ShapeDtypeStruct((1, 1), jnp.float32),
        ],
        compiler_params=pltpu.CompilerParams(
            dimension_semantics=("arbitrary",)),
    )(idx2, inputs, embedding_weight)
    loss = loss.reshape(())
    perp = perp.reshape(())
    return (loss, q, perp, enc)
